# baseline (speedup 1.0000x reference)
import functools

import jax
import jax.numpy as jnp
from jax import lax
from jax.experimental import pallas as pl
from jax.experimental.pallas import tpu as pltpu
from jax.experimental.pallas import tpu_sc as plsc

NUM_FIELDS = 26
BATCH = 16384
FIELD_SIZE = 100000
TOTAL_ROWS = NUM_FIELDS * FIELD_SIZE
LANES = 16
NUM_WORKERS = 32
BPW = BATCH // NUM_WORKERS
JV = BPW // LANES
N_EL = NUM_FIELDS * BPW

CALL_SUBS = (((0, 6), (6, 8)), ((14, 6), (20, 6)))


def _round_up(n, m):
    return (n + m - 1) // m * m


def _slice_spec(f0, nf):
    lo, hi = f0 * FIELD_SIZE, (f0 + nf) * FIELD_SIZE
    size = _round_up(hi - lo, 1024)
    start = min(lo, TOTAL_ROWS - size)
    return start, size, lo - start

_mesh = plsc.VectorSubcoreMesh(core_axis_name="c", subcore_axis_name="s")


def _make_call_kernel(subs, add_bias, accumulate):
    specs = [(f0, nf) + _slice_spec(f0, nf) for f0, nf in subs]
    n_els = [nf * BPW for _, nf, _, _, _ in specs]

    scratch = []
    for n_el in n_els:
        scratch.append(pltpu.VMEM((n_el,), jnp.int32))
        scratch.append(pltpu.VMEM((n_el,), jnp.float32))
    scratch += [
        pltpu.VMEM((LANES,), jnp.float32),
        pltpu.VMEM((BPW,), jnp.float32),
        pltpu.SemaphoreType.DMA,
        pltpu.SemaphoreType.DMA,
    ]

    @functools.partial(
        pl.kernel,
        mesh=_mesh,
        out_type=jax.ShapeDtypeStruct((BATCH,), jnp.float32),
        scratch_types=scratch,
    )
    def _call_kernel(x_hbm, *args):
        ns = len(specs)
        w_hbms = args[:ns]
        b_hbm = args[ns]
        acc_hbm = args[ns + 1] if accumulate else None
        out_hbm = args[ns + 1 + (1 if accumulate else 0)]
        rest = args[ns + 2 + (1 if accumulate else 0):]
        xvs = rest[0:2 * ns:2]
        gaths = rest[1:2 * ns:2]
        bias_v = rest[2 * ns]
        out_v = rest[2 * ns + 1]
        sem_x = rest[2 * ns + 2]
        sem_g = rest[2 * ns + 3]

        wid = lax.axis_index("s") * 2 + lax.axis_index("c")
        base = wid * BPW

        x_copies = [
            pltpu.async_copy(
                x_hbm.at[pl.ds(wid * N_EL + f0 * BPW, n_el)], xv, sem_x)
            for (f0, _, _, _, _), xv, n_el in zip(specs, xvs, n_els)
        ]
        if add_bias:
            pltpu.sync_copy(b_hbm, bias_v)

        g_copies = []
        for (f0, nf, _, _, local_base), xv, gath, w_hbm, xc in zip(
                specs, xvs, gaths, w_hbms, x_copies):
            xc.wait()

            def off_body(v, _, xv=xv, local_base=local_base):
                off = local_base + (v // JV) * FIELD_SIZE
                xv[pl.ds(v * LANES, LANES)] = (xv[pl.ds(v * LANES, LANES)]
                                               + off)
                return 0

            lax.fori_loop(0, nf * JV, off_body, 0)
            g_copies.append(pltpu.async_copy(w_hbm.at[xv], gath, sem_g))

        if accumulate:
            pltpu.sync_copy(acc_hbm.at[pl.ds(base, BPW)], out_v)
        if add_bias:
            seed = bias_v[...]
        else:
            seed = jnp.zeros((LANES,), jnp.float32)

        for si, ((f0, nf, _, _, _), gath, gc) in enumerate(
                zip(specs, gaths, g_copies)):
            gc.wait()
            first = si == 0 and not accumulate

            def r_body(j, _, gath=gath, nf=nf, first=first):
                b0 = (2 * j) * LANES
                b1 = b0 + LANES
                if first:
                    init = (seed, seed)
                else:
                    init = (out_v[pl.ds(b0, LANES)], out_v[pl.ds(b1, LANES)])

                def a_body(f, accs, gath=gath, b0=b0, b1=b1):
                    a0, a1 = accs
                    return (a0 + gath[pl.ds(f * BPW + b0, LANES)],
                            a1 + gath[pl.ds(f * BPW + b1, LANES)])

                a0, a1 = lax.fori_loop(0, nf, a_body, init)
                out_v[pl.ds(b0, LANES)] = a0
                out_v[pl.ds(b1, LANES)] = a1
                return 0

            lax.fori_loop(0, JV // 2, r_body, 0)

        pltpu.sync_copy(out_v, out_hbm.at[pl.ds(base, BPW)])

    return _call_kernel


_CALL_KERNELS = [_make_call_kernel(subs, i == 0, i > 0)
                 for i, subs in enumerate(CALL_SUBS)]


def kernel(x, fc_weight, bias):
    xp = (x.astype(jnp.int32)
          .reshape(NUM_WORKERS, BPW, NUM_FIELDS)
          .transpose(0, 2, 1)
          .reshape(-1))
    b16 = jnp.broadcast_to(bias.astype(jnp.float32), (LANES,))

    total = None
    dep = None
    for subs, ck in zip(CALL_SUBS, _CALL_KERNELS):
        sl2d = []
        for sj, (f0, nf) in enumerate(subs):
            start, size, _ = _slice_spec(f0, nf)
            s = lax.slice(fc_weight, (start, 0), (start + size, 1))
            if dep is not None:
                s = s + dep
            sl2d.append(s)
        bar = lax.optimization_barrier(tuple(sl2d))
        ws = [b.reshape(-1) for b in bar]
        dep = bar[0][:1, :] * 0.0
        if total is None:
            total = ck(xp, *ws, b16)
        else:
            total = ck(xp, *ws, b16, total)
    return total.reshape(BATCH, 1)


# speedup vs baseline: 1.0251x; 1.0251x over previous
import functools

import jax
import jax.numpy as jnp
from jax import lax
from jax.experimental import pallas as pl
from jax.experimental.pallas import tpu as pltpu
from jax.experimental.pallas import tpu_sc as plsc

NUM_FIELDS = 26
BATCH = 16384
FIELD_SIZE = 100000
TOTAL_ROWS = NUM_FIELDS * FIELD_SIZE
LANES = 16
NUM_WORKERS = 32
BPW = BATCH // NUM_WORKERS
JV = BPW // LANES
N_EL = NUM_FIELDS * BPW

CALL_SUBS = (((0, 6), (6, 6)), ((12, 6), (18, 8)))


def _round_up(n, m):
    return (n + m - 1) // m * m


def _slice_spec(f0, nf):
    lo, hi = f0 * FIELD_SIZE, (f0 + nf) * FIELD_SIZE
    size = _round_up(hi - lo, 1024)
    start = min(lo, TOTAL_ROWS - size)
    return start, size, lo - start

_mesh = plsc.VectorSubcoreMesh(core_axis_name="c", subcore_axis_name="s")


def _make_call_kernel(subs, add_bias, accumulate):
    specs = [(f0, nf) + _slice_spec(f0, nf) for f0, nf in subs]
    n_els = [nf * BPW for _, nf, _, _, _ in specs]

    scratch = []
    for n_el in n_els:
        scratch.append(pltpu.VMEM((n_el,), jnp.int32))
        scratch.append(pltpu.VMEM((n_el,), jnp.float32))
    scratch += [
        pltpu.VMEM((LANES,), jnp.float32),
        pltpu.VMEM((BPW,), jnp.float32),
        pltpu.SemaphoreType.DMA,
        pltpu.SemaphoreType.DMA,
    ]

    @functools.partial(
        pl.kernel,
        mesh=_mesh,
        out_type=jax.ShapeDtypeStruct((BATCH,), jnp.float32),
        scratch_types=scratch,
    )
    def _call_kernel(x_hbm, *args):
        ns = len(specs)
        w_hbms = args[:ns]
        b_hbm = args[ns]
        acc_hbm = args[ns + 1] if accumulate else None
        out_hbm = args[ns + 1 + (1 if accumulate else 0)]
        rest = args[ns + 2 + (1 if accumulate else 0):]
        xvs = rest[0:2 * ns:2]
        gaths = rest[1:2 * ns:2]
        bias_v = rest[2 * ns]
        out_v = rest[2 * ns + 1]
        sem_x = rest[2 * ns + 2]
        sem_g = rest[2 * ns + 3]

        wid = lax.axis_index("s") * 2 + lax.axis_index("c")
        base = wid * BPW

        x_copies = [
            pltpu.async_copy(
                x_hbm.at[pl.ds(wid * N_EL + f0 * BPW, n_el)], xv, sem_x)
            for (f0, _, _, _, _), xv, n_el in zip(specs, xvs, n_els)
        ]
        if add_bias:
            pltpu.sync_copy(b_hbm, bias_v)

        g_copies = []
        for (f0, nf, _, _, local_base), xv, gath, w_hbm, xc in zip(
                specs, xvs, gaths, w_hbms, x_copies):
            xc.wait()

            def off_body(v, _, xv=xv, local_base=local_base):
                off = local_base + (v // JV) * FIELD_SIZE
                xv[pl.ds(v * LANES, LANES)] = (xv[pl.ds(v * LANES, LANES)]
                                               + off)
                return 0

            lax.fori_loop(0, nf * JV, off_body, 0)
            g_copies.append(pltpu.async_copy(w_hbm.at[xv], gath, sem_g))

        if accumulate:
            pltpu.sync_copy(acc_hbm.at[pl.ds(base, BPW)], out_v)
        if add_bias:
            seed = bias_v[...]
        else:
            seed = jnp.zeros((LANES,), jnp.float32)

        for si, ((f0, nf, _, _, _), gath, gc) in enumerate(
                zip(specs, gaths, g_copies)):
            gc.wait()
            first = si == 0 and not accumulate

            def r_body(j, _, gath=gath, nf=nf, first=first):
                b0 = (2 * j) * LANES
                b1 = b0 + LANES
                if first:
                    init = (seed, seed)
                else:
                    init = (out_v[pl.ds(b0, LANES)], out_v[pl.ds(b1, LANES)])

                def a_body(f, accs, gath=gath, b0=b0, b1=b1):
                    a0, a1 = accs
                    return (a0 + gath[pl.ds(f * BPW + b0, LANES)],
                            a1 + gath[pl.ds(f * BPW + b1, LANES)])

                a0, a1 = lax.fori_loop(0, nf, a_body, init)
                out_v[pl.ds(b0, LANES)] = a0
                out_v[pl.ds(b1, LANES)] = a1
                return 0

            lax.fori_loop(0, JV // 2, r_body, 0)

        pltpu.sync_copy(out_v, out_hbm.at[pl.ds(base, BPW)])

    return _call_kernel


_CALL_KERNELS = [_make_call_kernel(subs, i == 0, i > 0)
                 for i, subs in enumerate(CALL_SUBS)]


def kernel(x, fc_weight, bias):
    xp = (x.astype(jnp.int32)
          .reshape(NUM_WORKERS, BPW, NUM_FIELDS)
          .transpose(0, 2, 1)
          .reshape(-1))
    b16 = jnp.broadcast_to(bias.astype(jnp.float32), (LANES,))

    total = None
    dep = None
    for subs, ck in zip(CALL_SUBS, _CALL_KERNELS):
        sl2d = []
        for sj, (f0, nf) in enumerate(subs):
            start, size, _ = _slice_spec(f0, nf)
            s = lax.slice(fc_weight, (start, 0), (start + size, 1))
            if dep is not None:
                s = s + dep
            sl2d.append(s)
        bar = lax.optimization_barrier(tuple(sl2d))
        ws = [b.reshape(-1) for b in bar]
        dep = bar[0][:1, :] * 0.0
        if total is None:
            total = ck(xp, *ws, b16)
        else:
            total = ck(xp, *ws, b16, total)
    return total.reshape(BATCH, 1)


# speedup vs baseline: 1.0464x; 1.0208x over previous
import functools

import jax
import jax.numpy as jnp
from jax import lax
from jax.experimental import pallas as pl
from jax.experimental.pallas import tpu as pltpu
from jax.experimental.pallas import tpu_sc as plsc

NUM_FIELDS = 26
BATCH = 16384
FIELD_SIZE = 100000
TOTAL_ROWS = NUM_FIELDS * FIELD_SIZE
LANES = 16
NUM_WORKERS = 32
BPW = BATCH // NUM_WORKERS
JV = BPW // LANES
N_EL = NUM_FIELDS * BPW

CALL_SUBS = (((0, 6), (6, 6)), ((12, 6), (18, 8)))


def _round_up(n, m):
    return (n + m - 1) // m * m


def _slice_spec(f0, nf):
    lo, hi = f0 * FIELD_SIZE, (f0 + nf) * FIELD_SIZE
    size = _round_up(hi - lo, 1024)
    start = min(lo, TOTAL_ROWS - size)
    return start, size, lo - start

_mesh = plsc.VectorSubcoreMesh(core_axis_name="c", subcore_axis_name="s")


def _make_call_kernel(subs, add_bias, accumulate):
    specs = [(f0, nf) + _slice_spec(f0, nf) for f0, nf in subs]
    n_els = [nf * BPW for _, nf, _, _, _ in specs]

    scratch = []
    for n_el in n_els:
        scratch.append(pltpu.VMEM((n_el,), jnp.int32))
        scratch.append(pltpu.VMEM((n_el,), jnp.float32))
    scratch += [
        pltpu.VMEM((LANES,), jnp.float32),
        pltpu.VMEM((BPW,), jnp.float32),
        pltpu.SemaphoreType.DMA,
        pltpu.SemaphoreType.DMA,
    ]

    @functools.partial(
        pl.kernel,
        mesh=_mesh,
        out_type=jax.ShapeDtypeStruct((BATCH,), jnp.float32),
        scratch_types=scratch,
    )
    def _call_kernel(x_hbm, *args):
        ns = len(specs)
        w_hbms = args[:ns]
        b_hbm = args[ns]
        acc_hbm = args[ns + 1] if accumulate else None
        out_hbm = args[ns + 1 + (1 if accumulate else 0)]
        rest = args[ns + 2 + (1 if accumulate else 0):]
        xvs = rest[0:2 * ns:2]
        gaths = rest[1:2 * ns:2]
        bias_v = rest[2 * ns]
        out_v = rest[2 * ns + 1]
        sem_x = rest[2 * ns + 2]
        sem_g = rest[2 * ns + 3]

        wid = lax.axis_index("s") * 2 + lax.axis_index("c")
        base = wid * BPW

        x_copies = [
            pltpu.async_copy(
                x_hbm.at[pl.ds(wid * N_EL + f0 * BPW, n_el)], xv, sem_x)
            for (f0, _, _, _, _), xv, n_el in zip(specs, xvs, n_els)
        ]
        if add_bias:
            pltpu.sync_copy(b_hbm, bias_v)

        g_copies = []
        for (f0, nf, _, _, local_base), xv, gath, w_hbm, xc in zip(
                specs, xvs, gaths, w_hbms, x_copies):
            xc.wait()

            def off_body(v, _, xv=xv, local_base=local_base):
                off = local_base + (v // JV) * FIELD_SIZE
                xv[pl.ds(v * LANES, LANES)] = (xv[pl.ds(v * LANES, LANES)]
                                               + off)
                return 0

            lax.fori_loop(0, nf * JV, off_body, 0)
            g_copies.append(pltpu.async_copy(w_hbm.at[xv], gath, sem_g))

        if accumulate:
            pltpu.sync_copy(acc_hbm.at[pl.ds(base, BPW)], out_v)
        if add_bias:
            seed = bias_v[...]
        else:
            seed = jnp.zeros((LANES,), jnp.float32)

        for si, ((f0, nf, _, _, _), gath, gc) in enumerate(
                zip(specs, gaths, g_copies)):
            gc.wait()
            first = si == 0 and not accumulate

            def r_body(j, _, gath=gath, nf=nf, first=first):
                b0 = (2 * j) * LANES
                b1 = b0 + LANES
                if first:
                    init = (seed, seed)
                else:
                    init = (out_v[pl.ds(b0, LANES)], out_v[pl.ds(b1, LANES)])

                def a_body(f, accs, gath=gath, b0=b0, b1=b1):
                    a0, a1 = accs
                    return (a0 + gath[pl.ds(f * BPW + b0, LANES)],
                            a1 + gath[pl.ds(f * BPW + b1, LANES)])

                a0, a1 = lax.fori_loop(0, nf, a_body, init)
                out_v[pl.ds(b0, LANES)] = a0
                out_v[pl.ds(b1, LANES)] = a1
                return 0

            lax.fori_loop(0, JV // 2, r_body, 0)

        pltpu.sync_copy(out_v, out_hbm.at[pl.ds(base, BPW)])

    return _call_kernel


_CALL_KERNELS = [_make_call_kernel(subs, i == 0, i > 0)
                 for i, subs in enumerate(CALL_SUBS)]


def kernel(x, fc_weight, bias):
    xp = (x.astype(jnp.int32)
          .reshape(NUM_WORKERS, BPW, NUM_FIELDS)
          .transpose(0, 2, 1)
          .reshape(-1))
    b16 = jnp.broadcast_to(bias.astype(jnp.float32), (LANES,))

    total = None
    dep = None
    for subs, ck in zip(CALL_SUBS, _CALL_KERNELS):
        sl2d = []
        for sj, (f0, nf) in enumerate(subs):
            start, size, _ = _slice_spec(f0, nf)
            s = lax.slice(fc_weight, (start, 0), (start + size, 1))
            if dep is not None and sj == 0:
                s = s + dep
            sl2d.append(s)
        bar = lax.optimization_barrier(tuple(sl2d))
        ws = [b.reshape(-1) for b in bar]
        dep = bar[0][:1, :] * 0.0
        if total is None:
            total = ck(xp, *ws, b16)
        else:
            total = ck(xp, *ws, b16, total)
    return total.reshape(BATCH, 1)
